# baseline (speedup 1.0000x reference)
import jax
import jax.numpy as jnp
from jax.experimental import pallas as pl
from jax.experimental.pallas import tpu as pltpu

N = 2048
F = 256
H = 3
C = 32
BM = 256
NB = N // BM
FS0 = 192
FN0 = 200
FIN0 = F + C
FIN = 2 * C


def _gat_kernel(x_ref, a_ref,
                We1_ref, b1_ref, We2_ref, b2_ref, We3_ref, b3_ref,
                Wf1_ref, bf1_ref, Wf2_ref, bf2_ref,
                out_ref,
                hx_s, fnT_s, e1T_s, e2T_s, aux_s, buf0, buf1, pmax, a_s):
    l = pl.program_id(0)
    b = pl.program_id(1)

    def compute_h(inp, We_ref):
        hx = jnp.dot(inp, We_ref[...], preferred_element_type=jnp.float32,
                     precision=jax.lax.Precision.HIGHEST)
        hx_s[...] = hx
        aux_s[0:1, :] = jnp.mean(hx, axis=0, keepdims=True)
        fnT = hx[:, FN0:FN0 + 8].T
        fnT_s[...] = fnT
        fnmax = jnp.max(fnT, axis=1, keepdims=True)
        d = fnT - fnmax
        e1T_s[...] = jnp.exp(d)
        e2T_s[...] = jnp.exp(0.2 * d)

    @pl.when(jnp.logical_and(l == 0, b == 0))
    def _():
        compute_h(x_ref[...], We1_ref)

    @pl.when(jnp.logical_and(l == 1, b == 0))
    def _():
        compute_h(buf0[...], We2_ref)

    @pl.when(jnp.logical_and(l == 2, b == 0))
    def _():
        compute_h(buf1[...], We3_ref)

    rows = pl.ds(b * BM, BM)

    @pl.when(l == 0)
    def _():
        a_s[rows, :] = a_ref[...]

    mask = a_s[rows, :] != 0
    acc = jnp.zeros((BM, C), jnp.float32)
    for k in range(H):
        fs_blk = hx_s[rows, FS0 + k:FS0 + k + 1]
        fnT = fnT_s[k:k + 1, :]
        fnmax = jnp.max(fnT, axis=1, keepdims=True)
        t = fs_blk + fnmax
        m = jnp.maximum(t, 0.2 * t)
        a1 = jnp.exp(t - m)
        a2 = jnp.exp(0.2 * t - m)
        s = fnT >= -fs_blk
        G = jnp.where(s, a1 * e1T_s[k:k + 1, :], a2 * e2T_s[k:k + 1, :])
        P = jnp.where(mask, G, 0.0)
        Q = jnp.dot(P, hx_s[:, 64 * k:64 * k + C + 1],
                    preferred_element_type=jnp.float32)
        num = Q[:, :C]
        den = Q[:, C:C + 1]
        r = jnp.where(den > 0, 1.0 / den, 0.0)
        acc = acc + jnp.where(den > 0, num * r,
                              aux_s[0:1, 64 * k:64 * k + C])

    ones_pad = jnp.ones((BM, C), jnp.float32)

    @pl.when(l == 0)
    def _():
        o = jnp.maximum(acc * (1.0 / H) + b1_ref[...], 0.0)
        buf0[rows, :] = jnp.concatenate([o, ones_pad], axis=1)

    @pl.when(l == 1)
    def _():
        o = jnp.maximum(acc * (1.0 / H) + b2_ref[...], 0.0)
        buf1[rows, :] = jnp.concatenate([o, ones_pad], axis=1)

    @pl.when(l == 2)
    def _():
        xo = jnp.maximum(acc * (1.0 / H) + b3_ref[...], 0.0)
        bmax = jnp.max(xo, axis=0, keepdims=True)
        prev = jnp.where(b == 0, -jnp.inf, pmax[...])
        pmax[...] = jnp.maximum(prev, bmax)

    @pl.when(jnp.logical_and(l == 2, b == NB - 1))
    def _():
        p = pmax[...]
        hf = jnp.maximum(
            jnp.dot(p, Wf1_ref[...], preferred_element_type=jnp.float32,
                    precision=jax.lax.Precision.HIGHEST)
            + bf1_ref[...], 0.0)
        out_ref[...] = (jnp.dot(hf, Wf2_ref[...],
                                preferred_element_type=jnp.float32,
                                precision=jax.lax.Precision.HIGHEST)
                        + bf2_ref[...])


def _fold(W, a_s, a_n, fin_ext):
    f = W.shape[0]
    sel = jnp.repeat(jnp.arange(H), C)
    bd_s = jnp.where(sel[:, None] == jnp.arange(H)[None, :],
                     a_s.reshape(-1)[:, None], 0.0)
    bd_n = jnp.where(sel[:, None] == jnp.arange(H)[None, :],
                     a_n.reshape(-1)[:, None], 0.0)
    Ws = jnp.dot(W, bd_s, precision=jax.lax.Precision.HIGHEST)
    Wn = jnp.dot(W, bd_n, precision=jax.lax.Precision.HIGHEST)
    blocks = []
    for k in range(H):
        blk = jnp.zeros((fin_ext, 64), jnp.float32)
        blk = blk.at[:f, :C].set(W[:, C * k:C * (k + 1)])
        blk = blk.at[f, C].set(1.0)
        blocks.append(blk)
    tail = jnp.zeros((fin_ext, 64), jnp.float32)
    tail = tail.at[:f, 0:H].set(Ws)
    tail = tail.at[:f, 8:8 + H].set(Wn)
    return jnp.concatenate(blocks + [tail], axis=1)


def kernel(x, W1, as1, an1, b1, W2, as2, an2, b2, W3, as3, an3, b3,
           Wf1, bf1, Wf2, bf2, a):
    a8 = a.astype(jnp.int8)
    x_ext = jnp.concatenate([x, jnp.ones((N, C), jnp.float32)], axis=1)
    We1 = _fold(W1, as1, an1, FIN0)
    We2 = _fold(W2, as2, an2, FIN)
    We3 = _fold(W3, as3, an3, FIN)

    def const(shape):
        return pl.BlockSpec(shape, lambda l, b: (0,) * len(shape))

    in_specs = [
        pl.BlockSpec((N, FIN0), lambda l, b: (0, 0)),
        pl.BlockSpec((BM, N),
                     lambda l, b: (jnp.where(l == 0, b, NB - 1), 0)),
        const((FIN0, 256)), const((1, C)),
        const((FIN, 256)), const((1, C)),
        const((FIN, 256)), const((1, C)),
        const((C, 2 * C)), const((1, 2 * C)),
        const((2 * C, 1)), const((1, 1)),
    ]
    out = pl.pallas_call(
        _gat_kernel,
        grid=(3, NB),
        in_specs=in_specs,
        out_specs=pl.BlockSpec((1, 1), lambda l, b: (0, 0)),
        out_shape=jax.ShapeDtypeStruct((1, 1), jnp.float32),
        scratch_shapes=[
            pltpu.VMEM((N, 256), jnp.float32),
            pltpu.VMEM((8, N), jnp.float32),
            pltpu.VMEM((8, N), jnp.float32),
            pltpu.VMEM((8, N), jnp.float32),
            pltpu.VMEM((8, 256), jnp.float32),
            pltpu.VMEM((N, 2 * C), jnp.float32),
            pltpu.VMEM((N, 2 * C), jnp.float32),
            pltpu.VMEM((1, C), jnp.float32),
            pltpu.VMEM((N, N), jnp.int8),
        ],
        compiler_params=pltpu.CompilerParams(
            dimension_semantics=("arbitrary", "arbitrary")),
    )(x_ext, a8, We1, b1.reshape(1, C), We2, b2.reshape(1, C),
      We3, b3.reshape(1, C), Wf1, bf1.reshape(1, 2 * C),
      Wf2, bf2.reshape(1, 1))
    return out
